# Initial kernel scaffold; baseline (speedup 1.0000x reference)
import jax
import jax.numpy as jnp
from jax.experimental import pallas as pl


def kernel(X, pair_v, pair_e, W, b, group_weight):
    raise NotImplementedError("write your pallas kernel here")


# speedup vs baseline: 4.8227x; 4.8227x over previous
import functools

import jax
import jax.numpy as jnp
from jax import lax
from jax.experimental import pallas as pl
from jax.experimental.pallas import tpu as pltpu
from jax.experimental.pallas import tpu_sc as plsc

N_V = 10000
N_E = 10000
N_PAIRS = 320000
D = 128
DA = 144
NC = 2
NS = 16
NW = NC * NS
PAIRS_PER_W = N_PAIRS // NW
CHUNK = 80
NCHUNK = PAIRS_PER_W // CHUNK
N_SEG = 10240
ROWS_PER_TILE = N_SEG // NS
ZROWS = 128

@functools.lru_cache(maxsize=1)
def _make_sc_segment_pass():
    mesh = plsc.VectorSubcoreMesh(core_axis_name="c", subcore_axis_name="s",
                                  num_cores=NC, num_subcores=NS)
    return pl.kernel(
        _sc_segment_body,
        out_type=jax.ShapeDtypeStruct((NC, N_SEG, DA), jnp.float32),
        mesh=mesh,
        scratch_types=[
            pltpu.VMEM_SHARED((N_SEG, DA), jnp.float32),
            pltpu.VMEM((ZROWS, DA), jnp.float32),
            pltpu.VMEM((CHUNK,), jnp.int32),
            pltpu.VMEM((CHUNK,), jnp.int32),
            pltpu.VMEM((CHUNK, DA), jnp.float32),
            pltpu.SemaphoreType.DMA,
        ],
        compiler_params=pltpu.CompilerParams(use_tc_tiling_on_sc=False),
    )


def _sc_segment_body(table, gidx, sidx, out, acc, zbuf, idx_g, idx_s, rows, sem):
    c = lax.axis_index("c")
    s = lax.axis_index("s")

    def zero_body(i, _):
        r = i // (DA // 16)
        col = i % (DA // 16)
        zbuf[r, pl.ds(col * 16, 16)] = jnp.zeros((16,), jnp.float32)
        return 0
    lax.fori_loop(0, ZROWS * (DA // 16), zero_body, 0)
    for j in range(ROWS_PER_TILE // ZROWS):
        pltpu.sync_copy(zbuf, acc.at[pl.ds(s * ROWS_PER_TILE + j * ZROWS, ZROWS)])
    plsc.subcore_barrier()

    base = (c * NS + s) * PAIRS_PER_W

    def chunk_body(t, _):
        off = base + t * CHUNK
        pltpu.sync_copy(gidx.at[pl.ds(off, CHUNK)], idx_g)
        pltpu.sync_copy(sidx.at[pl.ds(off, CHUNK)], idx_s)
        pltpu.async_copy(table.at[idx_g], rows, sem).wait()
        pltpu.sync_copy(rows, acc.at[idx_s], add=True)
        return 0
    lax.fori_loop(0, NCHUNK, chunk_body, 0)

    plsc.subcore_barrier()
    pltpu.sync_copy(acc.at[pl.ds(s * ROWS_PER_TILE, ROWS_PER_TILE)],
                    out.at[c, pl.ds(s * ROWS_PER_TILE, ROWS_PER_TILE)])


_BR = 1000


def _matmul_body(x_ref, w_ref, b_ref, o_ref):
    x = x_ref[...]
    w = w_ref[...]
    y = lax.dot_general(x, w, (((1,), (1,)), ((), ())),
                        preferred_element_type=jnp.float32)
    o_ref[...] = y + b_ref[...]


def _combine_body(p_ref, o_ref):
    p = p_ref[...]
    ssum = p[0] + p[1]
    deg = jnp.maximum(ssum[:, D:D + 1], 1.0)
    col = lax.broadcasted_iota(jnp.int32, (_BR, DA), 1)
    o_ref[...] = jnp.where(col < D, ssum / deg,
                           jnp.where(col == D, 1.0, 0.0))


def _finalize_body(p_ref, o_ref):
    p = p_ref[...]
    ssum = p[0] + p[1]
    deg = jnp.maximum(ssum[:, D:D + 1], 1.0)
    o_ref[...] = jnp.maximum(ssum[:, :D] / deg, 0.0)


def kernel(X, pair_v, pair_e, W, b, group_weight):
    del group_weight

    xt = pl.pallas_call(
        _matmul_body,
        grid=(N_V // _BR,),
        in_specs=[
            pl.BlockSpec((_BR, D), lambda i: (i, 0)),
            pl.BlockSpec((D, D), lambda i: (0, 0)),
            pl.BlockSpec((1, D), lambda i: (0, 0)),
        ],
        out_specs=pl.BlockSpec((_BR, D), lambda i: (i, 0)),
        out_shape=jax.ShapeDtypeStruct((N_V, D), jnp.float32),
    )(X, W, b.reshape(1, D))

    aug = jnp.concatenate(
        [xt, jnp.ones((N_V, 1), jnp.float32), jnp.zeros((N_V, DA - D - 1), jnp.float32)],
        axis=1)

    sc_pass = _make_sc_segment_pass()
    partial_e = sc_pass(aug, pair_v, pair_e)

    e_feat_aug = pl.pallas_call(
        _combine_body,
        grid=(N_E // _BR,),
        in_specs=[pl.BlockSpec((NC, _BR, DA), lambda i: (0, i, 0))],
        out_specs=pl.BlockSpec((_BR, DA), lambda i: (i, 0)),
        out_shape=jax.ShapeDtypeStruct((N_E, DA), jnp.float32),
    )(partial_e)

    partial_v = sc_pass(e_feat_aug, pair_e, pair_v)

    out = pl.pallas_call(
        _finalize_body,
        grid=(N_V // _BR,),
        in_specs=[pl.BlockSpec((NC, _BR, DA), lambda i: (0, i, 0))],
        out_specs=pl.BlockSpec((_BR, D), lambda i: (i, 0)),
        out_shape=jax.ShapeDtypeStruct((N_V, D), jnp.float32),
    )(partial_v)

    return out


# speedup vs baseline: 10.0726x; 2.0886x over previous
import functools

import jax
import jax.numpy as jnp
from jax import lax
from jax.experimental import pallas as pl
from jax.experimental.pallas import tpu as pltpu
from jax.experimental.pallas import tpu_sc as plsc

N_V = 10000
N_E = 10000
N_PAIRS = 320000
D = 128
DA = 144
NC = 2
NS = 16
NW = NC * NS
PAIRS_PER_W = N_PAIRS // NW
CHUNK = 80
NCHUNK = PAIRS_PER_W // CHUNK
N_SEG = 10240
ROWS_PER_TILE = N_SEG // NS
ZROWS = 64
NROW = 2
NIDX = 4
STEADY0 = 2
STEADYN = (NCHUNK - 1 - STEADY0 - 2) // NIDX


@functools.lru_cache(maxsize=1)
def _make_sc_segment_pass():
    mesh = plsc.VectorSubcoreMesh(core_axis_name="c", subcore_axis_name="s",
                                  num_cores=NC, num_subcores=NS)
    return pl.kernel(
        _sc_segment_body,
        out_type=jax.ShapeDtypeStruct((NC, N_SEG, DA), jnp.float32),
        mesh=mesh,
        scratch_types=[
            pltpu.VMEM_SHARED((N_SEG, DA), jnp.float32),
            pltpu.VMEM((ZROWS, DA), jnp.float32),
            pltpu.VMEM((NIDX, CHUNK), jnp.int32),
            pltpu.VMEM((NIDX, CHUNK), jnp.int32),
            pltpu.VMEM((NROW, CHUNK, DA), jnp.float32),
            *([pltpu.SemaphoreType.DMA] * (NIDX + 2 * NROW)),
        ],
        compiler_params=pltpu.CompilerParams(use_tc_tiling_on_sc=False),
    )


def _sc_segment_body(table, gidx, sidx, out, acc, zbuf, idx_g, idx_s, rows, *sems):
    sem_i = sems[:NIDX]
    sem_g = sems[NIDX:NIDX + NROW]
    sem_s = sems[NIDX + NROW:]
    c = lax.axis_index("c")
    s = lax.axis_index("s")
    wid = c * NS + s

    def zero_body(i, _):
        r = i // (DA // 16)
        col = i % (DA // 16)
        zbuf[r, pl.ds(col * 16, 16)] = jnp.zeros((16,), jnp.float32)
        return 0
    lax.fori_loop(0, ZROWS * (DA // 16), zero_body, 0)
    for j in range(ROWS_PER_TILE // ZROWS):
        pltpu.sync_copy(zbuf, acc.at[pl.ds(s * ROWS_PER_TILE + j * ZROWS, ZROWS)])
    plsc.subcore_barrier()

    def start_idx(t, bi):
        pltpu.async_copy(gidx.at[wid, t], idx_g.at[bi], sem_i[bi])
        pltpu.async_copy(sidx.at[wid, t], idx_s.at[bi], sem_i[bi])

    def wait_idx(bi):
        pltpu.make_async_copy(gidx.at[0, 0], idx_g.at[bi], sem_i[bi]).wait()
        pltpu.make_async_copy(sidx.at[0, 0], idx_s.at[bi], sem_i[bi]).wait()

    def start_gather(bi, br):
        pltpu.async_copy(table.at[idx_g.at[bi]], rows.at[br], sem_g[br])

    def wait_gather(bi, br):
        pltpu.make_async_copy(table.at[idx_g.at[bi]], rows.at[br], sem_g[br]).wait()

    def start_scatter(bi, br):
        pltpu.async_copy(rows.at[br], acc.at[idx_s.at[bi]], sem_s[br], add=True)

    def wait_scatter(bi, br):
        pltpu.make_async_copy(rows.at[br], acc.at[idx_s.at[bi]], sem_s[br]).wait()

    def step(t, bi, br, first=False, last=False):
        obi = (bi - 1) % NIDX
        obr = (br - 1) % NROW
        wait_idx(bi)
        if not first:
            wait_scatter(bi, br)
        start_gather(bi, br)
        if not last:
            start_idx(t + 1, (bi + 1) % NIDX)
        if not first:
            wait_gather(obi, obr)
            start_scatter(obi, obr)

    start_idx(0, 0)
    step(0, 0, 0, first=True)
    wait_idx(1)
    wait_gather(0, 0)
    start_scatter(0, 0)
    start_gather(1, 1)
    start_idx(2, 2)

    def group_body(g, _):
        for j in range(NIDX):
            t = STEADY0 + g * NIDX + j
            bi = (STEADY0 + j) % NIDX
            br = (STEADY0 + j) % NROW
            step(t, bi, br)
        return 0
    lax.fori_loop(0, STEADYN, group_body, 0)

    for t in range(STEADY0 + NIDX * STEADYN, NCHUNK):
        step(t, t % NIDX, t % NROW, last=(t == NCHUNK - 1))

    tl = NCHUNK - 1
    wait_gather(tl % NIDX, tl % NROW)
    start_scatter(tl % NIDX, tl % NROW)
    wait_scatter((tl - 1) % NIDX, (tl - 1) % NROW)
    wait_scatter(tl % NIDX, tl % NROW)

    plsc.subcore_barrier()
    pltpu.sync_copy(acc.at[pl.ds(s * ROWS_PER_TILE, ROWS_PER_TILE)],
                    out.at[c, pl.ds(s * ROWS_PER_TILE, ROWS_PER_TILE)])


_BR = 1000


def _matmul_body(x_ref, w_ref, b_ref, o_ref):
    x = x_ref[...]
    w = w_ref[...]
    y = lax.dot_general(x, w, (((1,), (1,)), ((), ())),
                        preferred_element_type=jnp.float32)
    o_ref[...] = y + b_ref[...]


def _combine_body(p_ref, o_ref):
    p = p_ref[...]
    ssum = p[0] + p[1]
    deg = jnp.maximum(ssum[:, D:D + 1], 1.0)
    col = lax.broadcasted_iota(jnp.int32, (_BR, DA), 1)
    o_ref[...] = jnp.where(col < D, ssum / deg,
                           jnp.where(col == D, 1.0, 0.0))


def _finalize_body(p_ref, o_ref):
    p = p_ref[...]
    ssum = p[0] + p[1]
    deg = jnp.maximum(ssum[:, D:D + 1], 1.0)
    o_ref[...] = jnp.maximum(ssum[:, :D] / deg, 0.0)


def kernel(X, pair_v, pair_e, W, b, group_weight):
    del group_weight

    xt = pl.pallas_call(
        _matmul_body,
        grid=(N_V // _BR,),
        in_specs=[
            pl.BlockSpec((_BR, D), lambda i: (i, 0)),
            pl.BlockSpec((D, D), lambda i: (0, 0)),
            pl.BlockSpec((1, D), lambda i: (0, 0)),
        ],
        out_specs=pl.BlockSpec((_BR, D), lambda i: (i, 0)),
        out_shape=jax.ShapeDtypeStruct((N_V, D), jnp.float32),
    )(X, W, b.reshape(1, D))

    aug = jnp.concatenate(
        [xt, jnp.ones((N_V, 1), jnp.float32), jnp.zeros((N_V, DA - D - 1), jnp.float32)],
        axis=1)

    pv3 = pair_v.reshape(NW, NCHUNK, CHUNK)
    pe3 = pair_e.reshape(NW, NCHUNK, CHUNK)

    sc_pass = _make_sc_segment_pass()
    partial_e = sc_pass(aug, pv3, pe3)

    e_feat_aug = pl.pallas_call(
        _combine_body,
        grid=(N_E // _BR,),
        in_specs=[pl.BlockSpec((NC, _BR, DA), lambda i: (0, i, 0))],
        out_specs=pl.BlockSpec((_BR, DA), lambda i: (i, 0)),
        out_shape=jax.ShapeDtypeStruct((N_E, DA), jnp.float32),
    )(partial_e)

    partial_v = sc_pass(e_feat_aug, pe3, pv3)

    out = pl.pallas_call(
        _finalize_body,
        grid=(N_V // _BR,),
        in_specs=[pl.BlockSpec((NC, _BR, DA), lambda i: (0, i, 0))],
        out_specs=pl.BlockSpec((_BR, D), lambda i: (i, 0)),
        out_shape=jax.ShapeDtypeStruct((N_V, D), jnp.float32),
    )(partial_v)

    return out


# speedup vs baseline: 10.5855x; 1.0509x over previous
import functools

import jax
import jax.numpy as jnp
from jax import lax
from jax.experimental import pallas as pl
from jax.experimental.pallas import tpu as pltpu
from jax.experimental.pallas import tpu_sc as plsc

N_V = 10000
N_E = 10000
N_PAIRS = 320000
D = 128
FH = 64
WB = FH + 16
NC = 2
NS = 16
CHUNK = 80
NCHUNK = N_PAIRS // NS // CHUNK
N_SEG = 10240
ROWS_PER_TILE = N_SEG // NS
NROW = 4
NIDX = 8
STEADY0 = 8
TAIL = 2
STEADYN = (NCHUNK - STEADY0 - TAIL) // NIDX
EROWS = N_E // NS // 5


def _sc_pass_body(final, table, gidx, sidx, zf, zd, out, acc, dacc,
                  idx_g, idx_s, rows, ones, cbuf, dbuf, cobuf, *sems):
    sem_i = sems[:NIDX]
    sem_g = sems[NIDX:NIDX + NROW]
    sem_s = sems[NIDX + NROW:]
    c = lax.axis_index("c")
    s = lax.axis_index("s")

    pltpu.sync_copy(zf, acc.at[pl.ds(s * ROWS_PER_TILE, ROWS_PER_TILE)])
    pltpu.sync_copy(zd, dacc.at[pl.ds(s * ROWS_PER_TILE, ROWS_PER_TILE)])

    def ones_body(r, _):
        ones[r, pl.ds(0, 16)] = jnp.ones((16,), jnp.float32)
        return 0
    lax.fori_loop(0, CHUNK, ones_body, 0)
    plsc.subcore_barrier()

    def start_idx(t, bi):
        pltpu.async_copy(gidx.at[c, s, t], idx_g.at[bi], sem_i[bi])
        pltpu.async_copy(sidx.at[s, t], idx_s.at[bi], sem_i[bi])

    def wait_idx(bi):
        pltpu.make_async_copy(gidx.at[0, 0, 0], idx_g.at[bi], sem_i[bi]).wait()
        pltpu.make_async_copy(sidx.at[0, 0], idx_s.at[bi], sem_i[bi]).wait()

    def start_gather(bi, br):
        pltpu.async_copy(table.at[idx_g.at[bi]], rows.at[br], sem_g[br])

    def wait_gather(bi, br):
        pltpu.make_async_copy(table.at[idx_g.at[bi]], rows.at[br], sem_g[br]).wait()

    def start_scatter(bi, br):
        pltpu.async_copy(rows.at[br], acc.at[idx_s.at[bi]], sem_s[br], add=True)
        pltpu.async_copy(ones, dacc.at[idx_s.at[bi]], sem_s[br], add=True)

    def wait_scatter(bi, br):
        pltpu.make_async_copy(rows.at[br], acc.at[idx_s.at[bi]], sem_s[br]).wait()
        pltpu.make_async_copy(ones, dacc.at[idx_s.at[bi]], sem_s[br]).wait()

    def step(t, bi, br, wait_s=True, retire=True, prefetch=True):
        wait_idx(bi)
        if wait_s:
            wait_scatter(bi, br)
        start_gather(bi, br)
        if prefetch:
            start_idx(t + 1, (bi + 1) % NIDX)
        if retire:
            obi = (bi - 1) % NIDX
            obr = (br - 1) % NROW
            wait_gather(obi, obr)
            start_scatter(obi, obr)

    start_idx(0, 0)
    for t in range(STEADY0):
        step(t, t % NIDX, t % NROW, wait_s=(t >= NROW), retire=(t >= 1))

    def group_body(g, _):
        for j in range(NIDX):
            t = STEADY0 + g * NIDX + j
            step(t, (STEADY0 + j) % NIDX, (STEADY0 + j) % NROW)
        return 0
    lax.fori_loop(0, STEADYN, group_body, 0)

    for t in range(NCHUNK - TAIL, NCHUNK):
        step(t, t % NIDX, t % NROW, prefetch=(t < NCHUNK - 1))

    tl = NCHUNK - 1
    wait_gather(tl % NIDX, tl % NROW)
    start_scatter(tl % NIDX, tl % NROW)
    for b in range(NROW):
        wait_scatter(0, b)

    plsc.subcore_barrier()

    def blk_body(i, _):
        r0 = s * (5 * EROWS) + i * EROWS
        pltpu.sync_copy(acc.at[pl.ds(r0, EROWS)], cbuf)
        pltpu.sync_copy(dacc.at[pl.ds(r0, EROWS)], dbuf)

        def row_body(r, _):
            deg = jnp.maximum(dbuf[r, pl.ds(0, 16)], 1.0)
            inv = 1.0 / deg
            for j in range(FH // 16):
                v = cbuf[r, pl.ds(16 * j, 16)] * inv
                if final:
                    v = jnp.maximum(v, 0.0)
                cobuf[r, pl.ds(16 * j, 16)] = v
            return 0
        lax.fori_loop(0, EROWS, row_body, 0)

        if final:
            pltpu.sync_copy(cobuf, out.at[pl.ds(r0, EROWS), pl.ds(c * FH, FH)])
        else:
            pltpu.sync_copy(cobuf, out.at[c, pl.ds(r0, EROWS)])
        return 0
    lax.fori_loop(0, 5, blk_body, 0)


@functools.lru_cache(maxsize=2)
def _make_sc_pass(final):
    mesh = plsc.VectorSubcoreMesh(core_axis_name="c", subcore_axis_name="s",
                                  num_cores=NC, num_subcores=NS)
    if final:
        out_type = jax.ShapeDtypeStruct((N_V, D), jnp.float32)
    else:
        out_type = jax.ShapeDtypeStruct((NC, N_SEG, FH), jnp.float32)
    return pl.kernel(
        functools.partial(_sc_pass_body, final),
        out_type=out_type,
        mesh=mesh,
        scratch_types=[
            pltpu.VMEM_SHARED((N_SEG, FH), jnp.float32),
            pltpu.VMEM_SHARED((N_SEG, 16), jnp.float32),
            pltpu.VMEM((NIDX, CHUNK), jnp.int32),
            pltpu.VMEM((NIDX, CHUNK), jnp.int32),
            pltpu.VMEM((NROW, CHUNK, FH), jnp.float32),
            pltpu.VMEM((CHUNK, 16), jnp.float32),
            pltpu.VMEM((EROWS, FH), jnp.float32),
            pltpu.VMEM((EROWS, 16), jnp.float32),
            pltpu.VMEM((EROWS, FH), jnp.float32),
            *([pltpu.SemaphoreType.DMA] * (NIDX + 2 * NROW)),
        ],
        compiler_params=pltpu.CompilerParams(use_tc_tiling_on_sc=False),
    )


_BR = 1000


def _matmul_body(x_ref, w_ref, b_ref, o_ref):
    x = x_ref[...]
    w = w_ref[...]
    y = lax.dot_general(x, w, (((1,), (1,)), ((), ())),
                        preferred_element_type=jnp.float32)
    y = y + b_ref[...]
    o_ref[...] = jnp.stack([y[:, :FH], y[:, FH:]], axis=0)


def kernel(X, pair_v, pair_e, W, b, group_weight):
    del group_weight

    xt2 = pl.pallas_call(
        _matmul_body,
        grid=(N_V // _BR,),
        in_specs=[
            pl.BlockSpec((_BR, D), lambda i: (i, 0)),
            pl.BlockSpec((D, D), lambda i: (0, 0)),
            pl.BlockSpec((1, D), lambda i: (0, 0)),
        ],
        out_specs=pl.BlockSpec((NC, _BR, FH), lambda i: (0, i, 0)),
        out_shape=jax.ShapeDtypeStruct((NC, N_V, FH), jnp.float32),
    )(X, W, b.reshape(1, D))

    pvg = jnp.stack([pair_v, pair_v + N_V]).reshape(NC, NS, NCHUNK, CHUNK)
    peg = jnp.stack([pair_e, pair_e + N_SEG]).reshape(NC, NS, NCHUNK, CHUNK)
    pes = pair_e.reshape(NS, NCHUNK, CHUNK)
    pvs = pair_v.reshape(NS, NCHUNK, CHUNK)
    zf = jnp.zeros((ROWS_PER_TILE, FH), jnp.float32)
    zd = jnp.zeros((ROWS_PER_TILE, 16), jnp.float32)

    ef = _make_sc_pass(False)(xt2.reshape(NC * N_V, FH), pvg, pes, zf, zd)
    out = _make_sc_pass(True)(ef.reshape(NC * N_SEG, FH), peg, pvs, zf, zd)
    return out
